# baseline (speedup 1.0000x reference)
import functools
import jax
import jax.numpy as jnp
from jax import lax
from jax.experimental import pallas as pl
from jax.experimental.pallas import tpu as pltpu
from jax.experimental.pallas import tpu_sc as plsc

_KEEP_RATIO = 0.5
_B = 8
_N = 512 * 512
_LANES = 128
_ROWS = _N // _LANES
_S = 8
_CR = _ROWS // _S
_K_ALL = max(1, int(_N * _KEEP_RATIO))

_NT = 4
_PER_TILE = _N // _NT
_CHUNK = 32768
_NCH = _PER_TILE // _CHUNK
_NB1 = 2048
_NB2 = 1024
_NB3 = 1024


def _tc_body(x_ref, z_ref, m_ref, nb_ref, acc_ref, a_ref):
    b = pl.program_id(0)
    s = pl.program_id(1)

    x = x_ref[0]
    z = z_ref[0]
    m = m_ref[0]

    bce = jnp.maximum(x, 0.0) - x * z + jnp.log1p(jnp.exp(-jnp.abs(x)))
    loss = bce * m
    posf = z * m
    negf = (1.0 - z) * m

    nb_ref[0] = jnp.where(negf > 0.0,
                          lax.bitcast_convert_type(loss, jnp.int32),
                          jnp.int32(-1))

    n_pos_p = jnp.sum(posf)
    s_pos_p = jnp.sum(loss * posf)
    n_neg_p = jnp.sum(negf)

    @pl.when(s == 0)
    def _init_acc():
        a_ref[0] = n_pos_p
        a_ref[1] = s_pos_p
        a_ref[2] = n_neg_p

    @pl.when(s != 0)
    def _accum():
        a_ref[0] += n_pos_p
        a_ref[1] += s_pos_p
        a_ref[2] += n_neg_p

    @pl.when(s == _S - 1)
    def _emit():
        acc_ref[b, 0] = a_ref[0]
        acc_ref[b, 1] = a_ref[1]
        acc_ref[b, 2] = a_ref[2]


_IOTA = lambda: lax.iota(jnp.int32, 16)


def _sc_body(nb_hbm, accs_hbm, out_hbm,
             hcnt, hsum, dbuf, pcnt, psum, sbi, sbf, tmpi, tmpf,
             pub_cnt, pub_sum, pub_scal):
    c = lax.axis_index("c")
    s = lax.axis_index("s")
    li = s // _NT
    p = s % _NT
    b = c * 4 + li
    base = b * _N + p * _PER_TILE
    is_leader = p == 0

    zi = jnp.zeros((16,), jnp.int32)
    zf = jnp.zeros((16,), jnp.float32)
    onesi = jnp.ones((16,), jnp.int32)
    lofs1 = _IOTA() * _NB1
    lofs2 = _IOTA() * _NB2
    lofs3 = _IOTA() * _NB3

    _DN = lax.GatherDimensionNumbers(offset_dims=(), collapsed_slice_dims=(0,),
                                     start_index_map=(0,))

    def take_i(x, idx):
        return lax.gather(x, idx[:, None], _DN, (1,),
                          mode=lax.GatherScatterMode.PROMISE_IN_BOUNDS)

    take_f = take_i

    def suffix_i(x):
        for off in (1, 2, 4, 8):
            sh = take_i(x, jnp.minimum(_IOTA() + off, 15))
            x = x + jnp.where(_IOTA() + off <= 15, sh, 0)
        return x

    def suffix_f(x):
        for off in (1, 2, 4, 8):
            sh = take_f(x, jnp.minimum(_IOTA() + off, 15))
            x = x + jnp.where(_IOTA() + off <= 15, sh, 0.0)
        return x

    def zero_hists(nwords, do_sum):
        def zbody(i, carry):
            hcnt[pl.ds(i * 16, 16)] = zi
            if do_sum:
                hsum[pl.ds(i * 16, 16)] = zf
            return carry
        lax.fori_loop(0, nwords // 16, zbody, jnp.int32(0))

    def data_pass(level, mval):
        def chunk(ch, carry):
            pltpu.sync_copy(nb_hbm.at[pl.ds(base + ch * _CHUNK, _CHUNK)],
                            dbuf)

            def vbody(i, carry2):
                for u in range(8):
                    off = i * 128 + u * 16
                    bits = dbuf[pl.ds(off, 16)]
                    if level == 1:
                        mask = bits >= 0
                        bucket = lax.shift_right_arithmetic(bits, 20) & (_NB1 - 1)
                        idx = lofs1 + bucket
                    elif level == 2:
                        mask = lax.shift_right_arithmetic(bits, 20) == mval
                        bucket = lax.shift_right_arithmetic(bits, 10) & (_NB2 - 1)
                        idx = lofs2 + bucket
                    else:
                        mask = lax.shift_right_arithmetic(bits, 10) == mval
                        bucket = bits & (_NB3 - 1)
                        idx = lofs3 + bucket
                    plsc.addupdate_scatter(hcnt, [idx], onesi, mask=mask)
                    if level != 3:
                        vals = lax.bitcast_convert_type(bits, jnp.float32)
                        plsc.addupdate_scatter(hsum, [idx], vals, mask=mask)
                return carry2
            lax.fori_loop(0, _CHUNK // 128, vbody, jnp.int32(0))
            return carry
        lax.fori_loop(0, _NCH, chunk, jnp.int32(0))

    def merge_publish(nb, do_sum):
        def mbody(i, carry):
            off = i * 16
            acc_c = hcnt[pl.ds(off, 16)]
            acc_s = hsum[pl.ds(off, 16)] if do_sum else None
            for l in range(1, 16):
                acc_c = acc_c + hcnt[pl.ds(l * nb + off, 16)]
                if do_sum:
                    acc_s = acc_s + hsum[pl.ds(l * nb + off, 16)]
            hcnt[pl.ds(off, 16)] = acc_c
            if do_sum:
                hsum[pl.ds(off, 16)] = acc_s
            return carry
        lax.fori_loop(0, nb // 16, mbody, jnp.int32(0))
        pltpu.sync_copy(hcnt.at[pl.ds(0, nb)], pub_cnt.at[li, p, pl.ds(0, nb)])
        if do_sum:
            pltpu.sync_copy(hsum.at[pl.ds(0, nb)],
                            pub_sum.at[li, p, pl.ds(0, nb)])

    def leader_fetch(nb, do_sum):
        for q in range(_NT):
            pltpu.sync_copy(pub_cnt.at[li, q, pl.ds(0, nb)],
                            pcnt.at[pl.ds(q * nb, nb)])
            if do_sum:
                pltpu.sync_copy(pub_sum.at[li, q, pl.ds(0, nb)],
                                psum.at[pl.ds(q * nb, nb)])

    def scan_level(nb, k_i, val_base):
        nv = nb // 16

        def sbody(i, carry):
            bL, cnt_ab, sum_ab, cum, cums = carry
            j0 = (nv - 1 - i) * 16
            cnt_v = pcnt[pl.ds(j0, 16)]
            for q in range(1, _NT):
                cnt_v = cnt_v + pcnt[pl.ds(q * nb + j0, 16)]
            if val_base is None:
                sum_v = psum[pl.ds(j0, 16)]
                for q in range(1, _NT):
                    sum_v = sum_v + psum[pl.ds(q * nb + j0, 16)]
            else:
                binvals = lax.bitcast_convert_type(val_base | (j0 + _IOTA()),
                                                   jnp.float32)
                sum_v = cnt_v.astype(jnp.float32) * binvals
            suf_c = suffix_i(cnt_v)
            suf_s = suffix_f(sum_v)
            vtot = take_i(suf_c, zi)
            stot = take_f(suf_s, zi)
            s_incl = cum + suf_c
            crossed = s_incl >= k_i
            nc = plsc.all_reduce_population_count(crossed)
            m = jnp.maximum(nc - 1, 0)
            hit = jnp.logical_and(cum < k_i, cum + vtot >= k_i)
            cnt_ab_new = take_i(s_incl, m) - take_i(cnt_v, m)
            sum_ab_new = cums + take_f(suf_s, m) - take_f(sum_v, m)
            bL = jnp.where(hit, j0 + m, bL)
            cnt_ab = jnp.where(hit, cnt_ab_new, cnt_ab)
            sum_ab = jnp.where(hit, sum_ab_new, sum_ab)
            return bL, cnt_ab, sum_ab, cum + vtot, cums + stot

        init = (zi, zi, zf, zi, zf)
        bL, cnt_ab, sum_ab, _, _ = lax.fori_loop(0, nv, sbody, init)
        return bL, cnt_ab, sum_ab

    def publish_splat(val):
        sbi[...] = val
        pltpu.sync_copy(sbi, pub_scal.at[li])

    def read_splat():
        pltpu.sync_copy(pub_scal.at[li], sbi)
        return take_i(sbi[...], zi)

    pltpu.sync_copy(accs_hbm.at[b], sbf)
    av = sbf[...]
    n_pos = take_f(av, zi)
    sum_pos = take_f(av, zi + 1)
    num_neg = take_f(av, zi + 2)
    n_remain = jnp.maximum(0.0, jnp.float32(_K_ALL) - n_pos)
    k_f = jnp.minimum(n_remain, num_neg)
    k_i = k_f.astype(jnp.int32)

    zero_hists(16 * _NB1, True)
    data_pass(1, jnp.int32(0))
    merge_publish(_NB1, True)
    plsc.subcore_barrier()

    @pl.when(is_leader)
    def _scan1():
        leader_fetch(_NB1, True)
    b1, cnt1, sum1 = scan_level(_NB1, k_i, None)

    @pl.when(is_leader)
    def _pub1():
        publish_splat(b1)
    plsc.subcore_barrier()
    m1 = read_splat()

    zero_hists(16 * _NB2, True)
    data_pass(2, m1)
    merge_publish(_NB2, True)
    plsc.subcore_barrier()

    @pl.when(is_leader)
    def _scan2():
        leader_fetch(_NB2, True)
    k2_i = k_i - cnt1
    b2, cnt2, sum2 = scan_level(_NB2, k2_i, None)

    @pl.when(is_leader)
    def _pub2():
        publish_splat((m1 << 10) | b2)
    plsc.subcore_barrier()
    m2 = read_splat()

    zero_hists(16 * _NB3, False)
    data_pass(3, m2)
    merge_publish(_NB3, False)
    plsc.subcore_barrier()

    @pl.when(is_leader)
    def _scan3():
        leader_fetch(_NB3, False)
    k3_i = k2_i - cnt2
    b3, cnt3, sum3 = scan_level(_NB3, k3_i, m2 << 10)

    @pl.when(is_leader)
    def _finish():
        t_bits = (m2 << 10) | b3
        t_val = lax.bitcast_convert_type(t_bits, jnp.float32)
        n_gt = (cnt1 + cnt2 + cnt3).astype(jnp.float32)
        s_gt = sum1 + sum2 + sum3
        sum_topk = jnp.where(k_i > 0, s_gt + (k_f - n_gt) * t_val, 0.0)
        cntk = n_pos + k_f
        img_loss = jnp.where(cntk > 0.0,
                             (sum_pos + sum_topk) / jnp.maximum(cntk, 1.0),
                             0.0)
        sbf[...] = img_loss
        pltpu.sync_copy(sbf, out_hbm.at[b])


def kernel(logits, targets, tissue_mask):
    xs = logits.reshape(_B, _ROWS, _LANES)
    zs = targets.reshape(_B, _ROWS, _LANES)
    ms = tissue_mask.reshape(_B, _ROWS, _LANES)

    in_spec = pl.BlockSpec((1, _CR, _LANES), lambda b, s: (b, s, 0))
    negbits, accs = pl.pallas_call(
        _tc_body,
        grid=(_B, _S),
        in_specs=[in_spec, in_spec, in_spec],
        out_specs=[pl.BlockSpec((1, _CR, _LANES), lambda b, s: (b, s, 0)),
                   pl.BlockSpec(memory_space=pltpu.SMEM)],
        out_shape=[jax.ShapeDtypeStruct((_B, _ROWS, _LANES), jnp.int32),
                   jax.ShapeDtypeStruct((_B, 16), jnp.float32)],
        scratch_shapes=[pltpu.SMEM((3,), jnp.float32)],
    )(xs, zs, ms)

    negflat = negbits.reshape(_B * _N)

    mesh = plsc.VectorSubcoreMesh(core_axis_name="c", subcore_axis_name="s")
    sc = functools.partial(
        pl.kernel,
        mesh=mesh,
        compiler_params=pltpu.CompilerParams(needs_layout_passes=False,
                                             use_tc_tiling_on_sc=False),
        out_type=jax.ShapeDtypeStruct((_B, 16), jnp.float32),
        scratch_types=[
            pltpu.VMEM((16 * _NB1,), jnp.int32),
            pltpu.VMEM((16 * _NB1,), jnp.float32),
            pltpu.VMEM((_CHUNK,), jnp.int32),
            pltpu.VMEM((_NT * _NB1,), jnp.int32),
            pltpu.VMEM((_NT * _NB1,), jnp.float32),
            pltpu.VMEM((16,), jnp.int32),
            pltpu.VMEM((16,), jnp.float32),
            pltpu.VMEM((16,), jnp.int32),
            pltpu.VMEM((16,), jnp.float32),
            pltpu.VMEM_SHARED((4, _NT, _NB1), jnp.int32),
            pltpu.VMEM_SHARED((4, _NT, _NB1), jnp.float32),
            pltpu.VMEM_SHARED((4, 16), jnp.int32),
        ],
    )(_sc_body)
    per_img = sc(negflat, accs)
    return jnp.sum(per_img[:, 0]) / jnp.float32(_B)


# speedup vs baseline: 1.0753x; 1.0753x over previous
import functools
import jax
import jax.numpy as jnp
from jax import lax
from jax.experimental import pallas as pl
from jax.experimental.pallas import tpu as pltpu
from jax.experimental.pallas import tpu_sc as plsc

_KEEP_RATIO = 0.5
_B = 8
_N = 512 * 512
_LANES = 128
_ROWS = _N // _LANES
_S = 8
_CR = _ROWS // _S
_K_ALL = max(1, int(_N * _KEEP_RATIO))

_NT = 4
_PER_TILE = _N // _NT
_CHUNK = 32768
_NCH = _PER_TILE // _CHUNK
_NB1 = 2048
_NB2 = 1024
_NB3 = 1024


def _tc_body(x_ref, z_ref, m_ref, nb_ref, acc_ref, a_ref):
    b = pl.program_id(0)
    s = pl.program_id(1)

    x = x_ref[0]
    z = z_ref[0]
    m = m_ref[0]

    bce = jnp.maximum(x, 0.0) - x * z + jnp.log1p(jnp.exp(-jnp.abs(x)))
    loss = bce * m
    posf = z * m
    negf = (1.0 - z) * m

    nb_ref[0] = jnp.where(negf > 0.0,
                          lax.bitcast_convert_type(loss, jnp.int32),
                          jnp.int32(-1))

    n_pos_p = jnp.sum(posf)
    s_pos_p = jnp.sum(loss * posf)
    n_neg_p = jnp.sum(negf)

    @pl.when(s == 0)
    def _init_acc():
        a_ref[0] = n_pos_p
        a_ref[1] = s_pos_p
        a_ref[2] = n_neg_p

    @pl.when(s != 0)
    def _accum():
        a_ref[0] += n_pos_p
        a_ref[1] += s_pos_p
        a_ref[2] += n_neg_p

    @pl.when(s == _S - 1)
    def _emit():
        acc_ref[b, 0] = a_ref[0]
        acc_ref[b, 1] = a_ref[1]
        acc_ref[b, 2] = a_ref[2]


_IOTA = lambda: lax.iota(jnp.int32, 16)


def _sc_body(nb_hbm, accs_hbm, out_hbm,
             hcnt, dbuf, pcnt, sbi, sbf,
             pub_cnt, pub_scal, pub_fin_i, pub_fin_f, sems):
    c = lax.axis_index("c")
    s = lax.axis_index("s")
    li = s // _NT
    p = s % _NT
    b = c * 4 + li
    base = b * _N + p * _PER_TILE
    is_leader = p == 0

    zi = jnp.zeros((16,), jnp.int32)
    zf = jnp.zeros((16,), jnp.float32)
    onesi = jnp.ones((16,), jnp.int32)
    lofs1 = _IOTA() * _NB1
    lofs2 = _IOTA() * _NB2
    lofs3 = _IOTA() * _NB3

    _DN = lax.GatherDimensionNumbers(offset_dims=(), collapsed_slice_dims=(0,),
                                     start_index_map=(0,))

    def take_i(x, idx):
        return lax.gather(x, idx[:, None], _DN, (1,),
                          mode=lax.GatherScatterMode.PROMISE_IN_BOUNDS)

    take_f = take_i

    def suffix_i(x):
        for off in (1, 2, 4, 8):
            sh = take_i(x, jnp.minimum(_IOTA() + off, 15))
            x = x + jnp.where(_IOTA() + off <= 15, sh, 0)
        return x

    def suffix_f(x):
        for off in (1, 2, 4, 8):
            sh = take_f(x, jnp.minimum(_IOTA() + off, 15))
            x = x + jnp.where(_IOTA() + off <= 15, sh, 0.0)
        return x

    def zero_hists(nwords):
        def zbody(i, carry):
            for u in range(8):
                hcnt[pl.ds(i * 128 + u * 16, 16)] = zi
            return carry
        lax.fori_loop(0, nwords // 128, zbody, jnp.int32(0))

    def fetch_chunk(ch, slot):
        return pltpu.async_copy(
            nb_hbm.at[pl.ds(base + ch * _CHUNK, _CHUNK)],
            dbuf.at[slot], sems.at[slot])

    def data_pass(level, mval, finalize=None):
        copies = [fetch_chunk(ch, ch % 2) for ch in range(_NCH)]
        acc = (zi, zf)

        for ch in range(_NCH):
            copies[ch].wait()

            def vbody(i, carry2):
                an, af = carry2
                for u in range(8):
                    off = i * 128 + u * 16
                    bits = dbuf[ch % 2, pl.ds(off, 16)]
                    if finalize is not None:
                        gt = bits > finalize
                        an = an + gt.astype(jnp.int32)
                        af = af + jnp.where(
                            gt, lax.bitcast_convert_type(bits, jnp.float32),
                            0.0)
                        continue
                    if level == 1:
                        mask = bits >= 0
                        bucket = lax.shift_right_arithmetic(bits, 20) & (_NB1 - 1)
                        idx = lofs1 + bucket
                    elif level == 2:
                        mask = lax.shift_right_arithmetic(bits, 20) == mval
                        bucket = lax.shift_right_arithmetic(bits, 10) & (_NB2 - 1)
                        idx = lofs2 + bucket
                    else:
                        mask = lax.shift_right_arithmetic(bits, 10) == mval
                        bucket = bits & (_NB3 - 1)
                        idx = lofs3 + bucket
                    plsc.addupdate_scatter(hcnt, [idx], onesi, mask=mask)
                return an, af
            acc = lax.fori_loop(0, _CHUNK // 128, vbody, acc)
        return acc

    def merge_publish(nb):
        def mbody(i, carry):
            off = i * 16
            acc_c = hcnt[pl.ds(off, 16)]
            for l in range(1, 16):
                acc_c = acc_c + hcnt[pl.ds(l * nb + off, 16)]
            hcnt[pl.ds(off, 16)] = acc_c
            return carry
        lax.fori_loop(0, nb // 16, mbody, jnp.int32(0))
        pltpu.sync_copy(hcnt.at[pl.ds(0, nb)], pub_cnt.at[li, p, pl.ds(0, nb)])

    def leader_fetch(nb):
        for q in range(_NT):
            pltpu.sync_copy(pub_cnt.at[li, q, pl.ds(0, nb)],
                            pcnt.at[pl.ds(q * nb, nb)])

    def scan_level(nb, k_i):
        nv = nb // 16

        def sbody(i, carry):
            bL, cnt_ab, cum = carry
            j0 = (nv - 1 - i) * 16
            cnt_v = pcnt[pl.ds(j0, 16)]
            for q in range(1, _NT):
                cnt_v = cnt_v + pcnt[pl.ds(q * nb + j0, 16)]
            suf_c = suffix_i(cnt_v)
            vtot = take_i(suf_c, zi)
            s_incl = cum + suf_c
            crossed = s_incl >= k_i
            nc = plsc.all_reduce_population_count(crossed)
            m = jnp.maximum(nc - 1, 0)
            hit = jnp.logical_and(cum < k_i, cum + vtot >= k_i)
            cnt_ab_new = take_i(s_incl, m) - take_i(cnt_v, m)
            bL = jnp.where(hit, j0 + m, bL)
            cnt_ab = jnp.where(hit, cnt_ab_new, cnt_ab)
            return bL, cnt_ab, cum + vtot

        init = (zi, zi, zi)
        bL, cnt_ab, _ = lax.fori_loop(0, nv, sbody, init)
        return bL, cnt_ab

    def publish_splat(val):
        sbi[...] = val
        pltpu.sync_copy(sbi, pub_scal.at[li])

    def read_splat():
        pltpu.sync_copy(pub_scal.at[li], sbi)
        return take_i(sbi[...], zi)

    pltpu.sync_copy(accs_hbm.at[b], sbf)
    av = sbf[...]
    n_pos = take_f(av, zi)
    sum_pos = take_f(av, zi + 1)
    num_neg = take_f(av, zi + 2)
    n_remain = jnp.maximum(0.0, jnp.float32(_K_ALL) - n_pos)
    k_f = jnp.minimum(n_remain, num_neg)
    k_i = k_f.astype(jnp.int32)

    zero_hists(16 * _NB1)
    data_pass(1, jnp.int32(0))
    merge_publish(_NB1)
    plsc.subcore_barrier()

    @pl.when(is_leader)
    def _scan1():
        leader_fetch(_NB1)
    b1, cnt1 = scan_level(_NB1, k_i)

    @pl.when(is_leader)
    def _pub1():
        publish_splat(b1)
    plsc.subcore_barrier()
    m1 = read_splat()

    zero_hists(16 * _NB2)
    data_pass(2, m1)
    merge_publish(_NB2)
    plsc.subcore_barrier()

    @pl.when(is_leader)
    def _scan2():
        leader_fetch(_NB2)
    k2_i = k_i - cnt1
    b2, cnt2 = scan_level(_NB2, k2_i)

    @pl.when(is_leader)
    def _pub2():
        publish_splat((m1 << 10) | b2)
    plsc.subcore_barrier()
    m2 = read_splat()

    zero_hists(16 * _NB3)
    data_pass(3, m2)
    merge_publish(_NB3)
    plsc.subcore_barrier()

    @pl.when(is_leader)
    def _scan3():
        leader_fetch(_NB3)
    k3_i = k2_i - cnt2
    b3, _ = scan_level(_NB3, k3_i)

    @pl.when(is_leader)
    def _pub3():
        publish_splat((m2 << 10) | b3)
    plsc.subcore_barrier()
    t_bits = read_splat()

    an, af = data_pass(0, None, finalize=t_bits)
    sbi[...] = an
    pltpu.sync_copy(sbi, pub_fin_i.at[li, p])
    sbf[...] = af
    pltpu.sync_copy(sbf, pub_fin_f.at[li, p])
    plsc.subcore_barrier()

    @pl.when(is_leader)
    def _finish():
        vn = zi
        vf = zf
        for q in range(_NT):
            pltpu.sync_copy(pub_fin_i.at[li, q], sbi)
            vn = vn + sbi[...]
            pltpu.sync_copy(pub_fin_f.at[li, q], sbf)
            vf = vf + sbf[...]
        n_gt = take_i(suffix_i(vn), zi).astype(jnp.float32)
        s_gt = take_f(suffix_f(vf), zi)
        t_val = lax.bitcast_convert_type(t_bits, jnp.float32)
        sum_topk = jnp.where(k_i > 0, s_gt + (k_f - n_gt) * t_val, 0.0)
        cntk = n_pos + k_f
        img_loss = jnp.where(cntk > 0.0,
                             (sum_pos + sum_topk) / jnp.maximum(cntk, 1.0),
                             0.0)
        sbf[...] = img_loss
        pltpu.sync_copy(sbf, out_hbm.at[b])


def kernel(logits, targets, tissue_mask):
    xs = logits.reshape(_B, _ROWS, _LANES)
    zs = targets.reshape(_B, _ROWS, _LANES)
    ms = tissue_mask.reshape(_B, _ROWS, _LANES)

    in_spec = pl.BlockSpec((1, _CR, _LANES), lambda b, s: (b, s, 0))
    negbits, accs = pl.pallas_call(
        _tc_body,
        grid=(_B, _S),
        in_specs=[in_spec, in_spec, in_spec],
        out_specs=[pl.BlockSpec((1, _CR, _LANES), lambda b, s: (b, s, 0)),
                   pl.BlockSpec(memory_space=pltpu.SMEM)],
        out_shape=[jax.ShapeDtypeStruct((_B, _ROWS, _LANES), jnp.int32),
                   jax.ShapeDtypeStruct((_B, 16), jnp.float32)],
        scratch_shapes=[pltpu.SMEM((3,), jnp.float32)],
    )(xs, zs, ms)

    negflat = negbits.reshape(_B * _N)

    mesh = plsc.VectorSubcoreMesh(core_axis_name="c", subcore_axis_name="s")
    sc = functools.partial(
        pl.kernel,
        mesh=mesh,
        compiler_params=pltpu.CompilerParams(needs_layout_passes=False,
                                             use_tc_tiling_on_sc=False),
        out_type=jax.ShapeDtypeStruct((_B, 16), jnp.float32),
        scratch_types=[
            pltpu.VMEM((16 * _NB1,), jnp.int32),
            pltpu.VMEM((2, _CHUNK), jnp.int32),
            pltpu.VMEM((_NT * _NB1,), jnp.int32),
            pltpu.VMEM((16,), jnp.int32),
            pltpu.VMEM((16,), jnp.float32),
            pltpu.VMEM_SHARED((4, _NT, _NB1), jnp.int32),
            pltpu.VMEM_SHARED((4, 16), jnp.int32),
            pltpu.VMEM_SHARED((4, _NT, 16), jnp.int32),
            pltpu.VMEM_SHARED((4, _NT, 16), jnp.float32),
            pltpu.SemaphoreType.DMA((2,)),
        ],
    )(_sc_body)
    per_img = sc(negflat, accs)
    return jnp.sum(per_img[:, 0]) / jnp.float32(_B)


# speedup vs baseline: 1.1239x; 1.0452x over previous
import functools
import jax
import jax.numpy as jnp
from jax import lax
from jax.experimental import pallas as pl
from jax.experimental.pallas import tpu as pltpu
from jax.experimental.pallas import tpu_sc as plsc

_KEEP_RATIO = 0.5
_B = 8
_N = 512 * 512
_LANES = 128
_ROWS = _N // _LANES
_S = 8
_CR = _ROWS // _S
_K_ALL = max(1, int(_N * _KEEP_RATIO))

_NT = 4
_PER_TILE = _N // _NT
_CHUNK = 32768
_NCH = _PER_TILE // _CHUNK
_NB1 = 2048
_NB2 = 1024
_NB3 = 1024


def _tc_body(x_ref, z_ref, m_ref, nb_ref, acc_ref, a_ref):
    b = pl.program_id(0)
    s = pl.program_id(1)

    x = x_ref[0]
    z = z_ref[0]
    m = m_ref[0]

    bce = jnp.maximum(x, 0.0) - x * z + jnp.log1p(jnp.exp(-jnp.abs(x)))
    loss = bce * m
    posf = z * m
    negf = (1.0 - z) * m

    nb_ref[0] = jnp.where(negf > 0.0,
                          lax.bitcast_convert_type(loss, jnp.int32),
                          jnp.int32(-1))

    n_pos_p = jnp.sum(posf)
    s_pos_p = jnp.sum(loss * posf)
    n_neg_p = jnp.sum(negf)

    @pl.when(s == 0)
    def _init_acc():
        a_ref[0] = n_pos_p
        a_ref[1] = s_pos_p
        a_ref[2] = n_neg_p

    @pl.when(s != 0)
    def _accum():
        a_ref[0] += n_pos_p
        a_ref[1] += s_pos_p
        a_ref[2] += n_neg_p

    @pl.when(s == _S - 1)
    def _emit():
        acc_ref[b, 0] = a_ref[0]
        acc_ref[b, 1] = a_ref[1]
        acc_ref[b, 2] = a_ref[2]


_IOTA = lambda: lax.iota(jnp.int32, 16)


def _sc_body(nb_hbm, accs_hbm, out_hbm,
             hcnt, dbuf, pcnt, sbi, sbf,
             pub_cnt, pub_scal, pub_fin_f, sems):
    c = lax.axis_index("c")
    s = lax.axis_index("s")
    li = s // _NT
    p = s % _NT
    b = c * 4 + li
    base = b * _N + p * _PER_TILE
    is_leader = p == 0

    zi = jnp.zeros((16,), jnp.int32)
    zf = jnp.zeros((16,), jnp.float32)
    onesi = jnp.ones((16,), jnp.int32)
    lofs1 = _IOTA() * _NB1
    lofs2 = _IOTA() * _NB2
    lofs3 = _IOTA() * _NB3

    _DN = lax.GatherDimensionNumbers(offset_dims=(), collapsed_slice_dims=(0,),
                                     start_index_map=(0,))

    def take_i(x, idx):
        return lax.gather(x, idx[:, None], _DN, (1,),
                          mode=lax.GatherScatterMode.PROMISE_IN_BOUNDS)

    take_f = take_i

    def suffix_i(x):
        for off in (1, 2, 4, 8):
            sh = take_i(x, jnp.minimum(_IOTA() + off, 15))
            x = x + jnp.where(_IOTA() + off <= 15, sh, 0)
        return x

    def suffix_f(x):
        for off in (1, 2, 4, 8):
            sh = take_f(x, jnp.minimum(_IOTA() + off, 15))
            x = x + jnp.where(_IOTA() + off <= 15, sh, 0.0)
        return x

    def zero_hists(nwords):
        def zbody(i, carry):
            for u in range(8):
                hcnt[pl.ds(i * 128 + u * 16, 16)] = zi
            return carry
        lax.fori_loop(0, nwords // 128, zbody, jnp.int32(0))

    def fetch_chunk(ch, slot):
        return pltpu.async_copy(
            nb_hbm.at[pl.ds(base + ch * _CHUNK, _CHUNK)],
            dbuf.at[slot], sems.at[slot])

    def data_pass(level, mval):
        copies = [fetch_chunk(ch, ch % 2) for ch in range(_NCH)]
        acc = zf

        for ch in range(_NCH):
            copies[ch].wait()

            def vbody(i, af):
                for u in range(8):
                    off = i * 128 + u * 16
                    bits = dbuf[ch % 2, pl.ds(off, 16)]
                    if level == 1:
                        mask = bits >= 0
                        bucket = lax.shift_right_arithmetic(bits, 20) & (_NB1 - 1)
                        idx = lofs1 + bucket
                    elif level == 2:
                        mask = lax.shift_right_arithmetic(bits, 20) == mval
                        bucket = lax.shift_right_arithmetic(bits, 10) & (_NB2 - 1)
                        idx = lofs2 + bucket
                    else:
                        mask = lax.shift_right_arithmetic(bits, 10) == mval
                        bucket = bits & (_NB3 - 1)
                        idx = lofs3 + bucket
                        above = bits >= ((mval + 1) << 10)
                        af = af + jnp.where(
                            above,
                            lax.bitcast_convert_type(bits, jnp.float32), 0.0)
                    plsc.addupdate_scatter(hcnt, [idx], onesi, mask=mask)
                return af
            acc = lax.fori_loop(0, _CHUNK // 128, vbody, acc)
        return acc

    def merge_publish(nb):
        def mbody(i, carry):
            off = i * 16
            acc_c = hcnt[pl.ds(off, 16)]
            for l in range(1, 16):
                acc_c = acc_c + hcnt[pl.ds(l * nb + off, 16)]
            hcnt[pl.ds(off, 16)] = acc_c
            return carry
        lax.fori_loop(0, nb // 16, mbody, jnp.int32(0))
        pltpu.sync_copy(hcnt.at[pl.ds(0, nb)], pub_cnt.at[li, p, pl.ds(0, nb)])

    def leader_fetch(nb):
        for q in range(_NT):
            pltpu.sync_copy(pub_cnt.at[li, q, pl.ds(0, nb)],
                            pcnt.at[pl.ds(q * nb, nb)])

    def scan_level(nb, k_i, val_base=None):
        nv = nb // 16

        def sbody(i, carry):
            bL, cnt_ab, sum_ab, cum, cums = carry
            j0 = (nv - 1 - i) * 16
            cnt_v = pcnt[pl.ds(j0, 16)]
            for q in range(1, _NT):
                cnt_v = cnt_v + pcnt[pl.ds(q * nb + j0, 16)]
            suf_c = suffix_i(cnt_v)
            vtot = take_i(suf_c, zi)
            s_incl = cum + suf_c
            crossed = s_incl >= k_i
            nc = plsc.all_reduce_population_count(crossed)
            m = jnp.maximum(nc - 1, 0)
            hit = jnp.logical_and(cum < k_i, cum + vtot >= k_i)
            cnt_ab_new = take_i(s_incl, m) - take_i(cnt_v, m)
            bL = jnp.where(hit, j0 + m, bL)
            cnt_ab = jnp.where(hit, cnt_ab_new, cnt_ab)
            if val_base is not None:
                binvals = lax.bitcast_convert_type(
                    (val_base << 10) | (j0 + _IOTA()), jnp.float32)
                sum_v = cnt_v.astype(jnp.float32) * binvals
                suf_s = suffix_f(sum_v)
                stot = take_f(suf_s, zi)
                sum_ab_new = cums + take_f(suf_s, m) - take_f(sum_v, m)
                sum_ab = jnp.where(hit, sum_ab_new, sum_ab)
                cums = cums + stot
            return bL, cnt_ab, sum_ab, cum + vtot, cums

        init = (zi, zi, zf, zi, zf)
        bL, cnt_ab, sum_ab, _, _ = lax.fori_loop(0, nv, sbody, init)
        return bL, cnt_ab, sum_ab

    def publish_splat(val):
        sbi[...] = val
        pltpu.sync_copy(sbi, pub_scal.at[li])

    def read_splat():
        pltpu.sync_copy(pub_scal.at[li], sbi)
        return take_i(sbi[...], zi)

    pltpu.sync_copy(accs_hbm.at[b], sbf)
    av = sbf[...]
    n_pos = take_f(av, zi)
    sum_pos = take_f(av, zi + 1)
    num_neg = take_f(av, zi + 2)
    n_remain = jnp.maximum(0.0, jnp.float32(_K_ALL) - n_pos)
    k_f = jnp.minimum(n_remain, num_neg)
    k_i = k_f.astype(jnp.int32)

    zero_hists(16 * _NB1)
    data_pass(1, jnp.int32(0))
    merge_publish(_NB1)
    plsc.subcore_barrier()

    @pl.when(is_leader)
    def _scan1():
        leader_fetch(_NB1)
    b1, cnt1, _ = scan_level(_NB1, k_i)

    @pl.when(is_leader)
    def _pub1():
        publish_splat(b1)
    plsc.subcore_barrier()
    m1 = read_splat()

    zero_hists(16 * _NB2)
    data_pass(2, m1)
    merge_publish(_NB2)
    plsc.subcore_barrier()

    @pl.when(is_leader)
    def _scan2():
        leader_fetch(_NB2)
    k2_i = k_i - cnt1
    b2, cnt2, _ = scan_level(_NB2, k2_i)

    @pl.when(is_leader)
    def _pub2():
        publish_splat((m1 << 10) | b2)
    plsc.subcore_barrier()
    m2 = read_splat()

    zero_hists(16 * _NB3)
    af = data_pass(3, m2)
    merge_publish(_NB3)
    sbf[...] = af
    pltpu.sync_copy(sbf, pub_fin_f.at[li, p])
    plsc.subcore_barrier()

    @pl.when(is_leader)
    def _scan3():
        leader_fetch(_NB3)
    k3_i = k2_i - cnt2
    b3, cnt3, sum3 = scan_level(_NB3, k3_i, val_base=m2)

    @pl.when(is_leader)
    def _finish():
        vf = zf
        for q in range(_NT):
            pltpu.sync_copy(pub_fin_f.at[li, q], sbf)
            vf = vf + sbf[...]
        s_gt = take_f(suffix_f(vf), zi) + sum3
        n_gt = (cnt1 + cnt2 + cnt3).astype(jnp.float32)
        t_bits = (m2 << 10) | b3
        t_val = lax.bitcast_convert_type(t_bits, jnp.float32)
        sum_topk = jnp.where(k_i > 0, s_gt + (k_f - n_gt) * t_val, 0.0)
        cntk = n_pos + k_f
        img_loss = jnp.where(cntk > 0.0,
                             (sum_pos + sum_topk) / jnp.maximum(cntk, 1.0),
                             0.0)
        sbf[...] = img_loss
        pltpu.sync_copy(sbf, out_hbm.at[b])


def kernel(logits, targets, tissue_mask):
    xs = logits.reshape(_B, _ROWS, _LANES)
    zs = targets.reshape(_B, _ROWS, _LANES)
    ms = tissue_mask.reshape(_B, _ROWS, _LANES)

    in_spec = pl.BlockSpec((1, _CR, _LANES), lambda b, s: (b, s, 0))
    negbits, accs = pl.pallas_call(
        _tc_body,
        grid=(_B, _S),
        in_specs=[in_spec, in_spec, in_spec],
        out_specs=[pl.BlockSpec((1, _CR, _LANES), lambda b, s: (b, s, 0)),
                   pl.BlockSpec(memory_space=pltpu.SMEM)],
        out_shape=[jax.ShapeDtypeStruct((_B, _ROWS, _LANES), jnp.int32),
                   jax.ShapeDtypeStruct((_B, 16), jnp.float32)],
        scratch_shapes=[pltpu.SMEM((3,), jnp.float32)],
    )(xs, zs, ms)

    negflat = negbits.reshape(_B * _N)

    mesh = plsc.VectorSubcoreMesh(core_axis_name="c", subcore_axis_name="s")
    sc = functools.partial(
        pl.kernel,
        mesh=mesh,
        compiler_params=pltpu.CompilerParams(needs_layout_passes=False,
                                             use_tc_tiling_on_sc=False),
        out_type=jax.ShapeDtypeStruct((_B, 16), jnp.float32),
        scratch_types=[
            pltpu.VMEM((16 * _NB1,), jnp.int32),
            pltpu.VMEM((2, _CHUNK), jnp.int32),
            pltpu.VMEM((_NT * _NB1,), jnp.int32),
            pltpu.VMEM((16,), jnp.int32),
            pltpu.VMEM((16,), jnp.float32),
            pltpu.VMEM_SHARED((4, _NT, _NB1), jnp.int32),
            pltpu.VMEM_SHARED((4, 16), jnp.int32),
            pltpu.VMEM_SHARED((4, _NT, 16), jnp.float32),
            pltpu.SemaphoreType.DMA((2,)),
        ],
    )(_sc_body)
    per_img = sc(negflat, accs)
    return jnp.sum(per_img[:, 0]) / jnp.float32(_B)
